# baseline (speedup 1.0000x reference)
import functools

import jax
import jax.numpy as jnp
from jax import lax
from jax.experimental import pallas as pl
from jax.experimental.pallas import tpu as pltpu
from jax.experimental.pallas import tpu_sc as plsc

_B, _C_IN, _N = 4, 128, 2048
_C_MID = 128
_C_E = 256
_K = 16
_Q = 512
_R = _B * _N
_LANES = 16
_W_PAIRS = 8
_W_IDX = _W_PAIRS * _K

_NEG = -3.0e38


def _tc_body(x_ref, xq_ref, w0_ref, w1T_ref, w2T_ref,
             b0c_ref, bb_ref, mm_ref, base_ref, gidx_ref):
    x = x_ref[...]
    net = jnp.maximum(
        jnp.dot(w0_ref[...], x, preferred_element_type=jnp.float32)
        + b0c_ref[...], 0.0)
    netq = jnp.maximum(
        jnp.dot(w0_ref[...], xq_ref[...], preferred_element_type=jnp.float32)
        + b0c_ref[...], 0.0)
    netT = jnp.transpose(netq)
    sq = jnp.sum(net * net, axis=0, keepdims=True)
    dots = jnp.dot(netT, net, preferred_element_type=jnp.float32)
    q0 = pl.program_id(0) * _Q
    qiota = lax.broadcasted_iota(jnp.int32, (_Q, _N), 0) + q0
    iota = lax.broadcasted_iota(jnp.int32, (_Q, _N), 1)
    s = jnp.where(iota == qiota, _NEG, 2.0 * dots - sq)
    mm = jnp.dot(netT, w1T_ref[...], preferred_element_type=jnp.float32)
    base = (jnp.dot(netT, w2T_ref[...], preferred_element_type=jnp.float32)
            + bb_ref[...] - mm)
    mm_ref[...] = mm
    base_ref[...] = base
    cols = []
    for t in range(_K):
        am = jnp.argmax(s, axis=1).astype(jnp.int32)[:, None]
        cols.append(am)
        if t < _K - 1:
            s = jnp.where(iota == am, _NEG, s)
    gidx_ref[...] = jnp.concatenate(cols, axis=1)


def _tc_stage(x_b, w0, w1T, w2T, b0c, bb):
    return pl.pallas_call(
        _tc_body,
        grid=(_N // _Q,),
        in_specs=[
            pl.BlockSpec((_C_IN, _N), lambda q: (0, 0)),
            pl.BlockSpec((_C_IN, _Q), lambda q: (0, q)),
            pl.BlockSpec((_C_MID, _C_IN), lambda q: (0, 0)),
            pl.BlockSpec((_C_MID, _C_E), lambda q: (0, 0)),
            pl.BlockSpec((_C_MID, _C_E), lambda q: (0, 0)),
            pl.BlockSpec((_C_MID, 1), lambda q: (0, 0)),
            pl.BlockSpec((1, _C_E), lambda q: (0, 0)),
        ],
        out_specs=[
            pl.BlockSpec((_Q, _C_E), lambda q: (q, 0)),
            pl.BlockSpec((_Q, _C_E), lambda q: (q, 0)),
            pl.BlockSpec((_Q, _K), lambda q: (q, 0)),
        ],
        out_shape=[
            jax.ShapeDtypeStruct((_N, _C_E), jnp.float32),
            jax.ShapeDtypeStruct((_N, _C_E), jnp.float32),
            jax.ShapeDtypeStruct((_N, _K), jnp.int32),
        ],
    )(x_b, x_b, w0, w1T, w2T, b0c, bb)


def _sc_stage(mm_rows, gidx_flat, base_rows):
    mesh = plsc.VectorSubcoreMesh(core_axis_name="core",
                                  subcore_axis_name="subcore")

    nrows = base_rows.shape[0]

    @functools.partial(
        pl.kernel,
        out_type=jax.ShapeDtypeStruct((nrows, _C_E), jnp.float32),
        mesh=mesh,
        scratch_types=[pltpu.VMEM((_W_IDX, _C_E), jnp.float32)],
    )
    def k(mm_hbm, gidx_hbm, base_hbm, out_hbm, rows_v):
        def body(i_vmem, base_vmem, o_vmem):
            pltpu.sync_copy(mm_hbm.at[i_vmem.at[0]], rows_v)

            @pl.loop(0, _W_PAIRS)
            def _(q):
                r0 = q * _K
                for ch in range(_C_E // _LANES):
                    sl = pl.ds(ch * _LANES, _LANES)
                    acc = rows_v[r0, sl]
                    for j in range(1, _K):
                        acc = jnp.maximum(acc, rows_v[r0 + j, sl])
                    o_vmem[q, sl] = jnp.maximum(acc + base_vmem[q, sl], 0.0)

        pltpu.emit_pipeline(
            body,
            grid=(nrows // _W_PAIRS,),
            in_specs=[
                pl.BlockSpec((1, _W_IDX), lambda i: (0, i)),
                pl.BlockSpec((_W_PAIRS, _C_E), lambda i: (i, 0)),
            ],
            out_specs=[pl.BlockSpec((_W_PAIRS, _C_E), lambda i: (i, 0))],
            core_axis_name=("core", "subcore"),
            dimension_semantics=(pltpu.PARALLEL,),
        )(gidx_hbm, base_hbm, out_hbm)

    return k(mm_rows, gidx_flat, base_rows)


def kernel(inputs, w0, b0, w1, b1, w2, b2):
    w1T = jnp.transpose(w1)
    w2T = jnp.transpose(w2)
    b0c = b0.reshape(_C_MID, 1)
    bb = (b1 + b2).reshape(1, _C_E)
    halves = []
    nh = _N // 2
    for b in range(_B):
        mm_b, base_b, gidx_b = _tc_stage(inputs[b], w0, w1T, w2T, b0c, bb)
        for h in range(2):
            halves.append(_sc_stage(
                mm_b,
                gidx_b[h * nh:(h + 1) * nh].reshape(1, nh * _K),
                base_b[h * nh:(h + 1) * nh]))
    y_rows = jnp.concatenate(halves).reshape(_B, _N, _C_E)
    y = jnp.transpose(y_rows, (0, 2, 1))
    return y.reshape(_B, _C_MID, 2 * _N)


# speedup vs baseline: 1.0099x; 1.0099x over previous
import functools

import jax
import jax.numpy as jnp
from jax import lax
from jax.experimental import pallas as pl
from jax.experimental.pallas import tpu as pltpu
from jax.experimental.pallas import tpu_sc as plsc

_B, _C_IN, _N = 4, 128, 2048
_C_MID = 128
_C_E = 256
_K = 16
_Q = 512
_R = _B * _N
_LANES = 16
_W_PAIRS = 8
_W_IDX = _W_PAIRS * _K

_NEG = -3.0e38


def _tc_body(xT_ref, x_ref, w0_ref, w0T_ref, w1T_ref, w2T_ref,
             b0c_ref, b0r_ref, bb_ref, mm_ref, base_ref, gidx_ref):
    x = x_ref[...]
    net = jnp.maximum(
        jnp.dot(w0_ref[...], x, preferred_element_type=jnp.float32)
        + b0c_ref[...], 0.0)
    netT = jnp.maximum(
        jnp.dot(xT_ref[...], w0T_ref[...], preferred_element_type=jnp.float32)
        + b0r_ref[...], 0.0)
    sq = jnp.sum(net * net, axis=0, keepdims=True)
    dots = jnp.dot(netT, net, preferred_element_type=jnp.float32)
    q0 = pl.program_id(0) * _Q
    qiota = lax.broadcasted_iota(jnp.int32, (_Q, _N), 0) + q0
    iota = lax.broadcasted_iota(jnp.int32, (_Q, _N), 1)
    s = jnp.where(iota == qiota, _NEG, 2.0 * dots - sq)
    mm = jnp.dot(netT, w1T_ref[...], preferred_element_type=jnp.float32)
    base = (jnp.dot(netT, w2T_ref[...], preferred_element_type=jnp.float32)
            + bb_ref[...] - mm)
    mm_ref[...] = mm
    base_ref[...] = base
    cols = []
    for t in range(_K):
        am = jnp.argmax(s, axis=1).astype(jnp.int32)[:, None]
        cols.append(am)
        if t < _K - 1:
            s = jnp.where(iota == am, _NEG, s)
    gidx_ref[...] = jnp.concatenate(cols, axis=1)


def _tc_stage(x_b, xT_b, w0, w0T, w1T, w2T, b0c, b0r, bb):
    return pl.pallas_call(
        _tc_body,
        grid=(_N // _Q,),
        in_specs=[
            pl.BlockSpec((_Q, _C_IN), lambda q: (q, 0)),
            pl.BlockSpec((_C_IN, _N), lambda q: (0, 0)),
            pl.BlockSpec((_C_MID, _C_IN), lambda q: (0, 0)),
            pl.BlockSpec((_C_IN, _C_MID), lambda q: (0, 0)),
            pl.BlockSpec((_C_MID, _C_E), lambda q: (0, 0)),
            pl.BlockSpec((_C_MID, _C_E), lambda q: (0, 0)),
            pl.BlockSpec((_C_MID, 1), lambda q: (0, 0)),
            pl.BlockSpec((1, _C_MID), lambda q: (0, 0)),
            pl.BlockSpec((1, _C_E), lambda q: (0, 0)),
        ],
        out_specs=[
            pl.BlockSpec((_Q, _C_E), lambda q: (q, 0)),
            pl.BlockSpec((_Q, _C_E), lambda q: (q, 0)),
            pl.BlockSpec((_Q, _K), lambda q: (q, 0)),
        ],
        out_shape=[
            jax.ShapeDtypeStruct((_N, _C_E), jnp.float32),
            jax.ShapeDtypeStruct((_N, _C_E), jnp.float32),
            jax.ShapeDtypeStruct((_N, _K), jnp.int32),
        ],
    )(xT_b, x_b, w0, w0T, w1T, w2T, b0c, b0r, bb)


def _sc_stage(mm_rows, gidx_flat, base_rows):
    mesh = plsc.VectorSubcoreMesh(core_axis_name="core",
                                  subcore_axis_name="subcore")

    nrows = base_rows.shape[0]

    @functools.partial(
        pl.kernel,
        out_type=jax.ShapeDtypeStruct((nrows, _C_E), jnp.float32),
        mesh=mesh,
        scratch_types=[pltpu.VMEM((_W_IDX, _C_E), jnp.float32)],
    )
    def k(mm_hbm, gidx_hbm, base_hbm, out_hbm, rows_v):
        def body(i_vmem, base_vmem, o_vmem):
            pltpu.sync_copy(mm_hbm.at[i_vmem.at[0]], rows_v)

            @pl.loop(0, _W_PAIRS)
            def _(q):
                r0 = q * _K
                for ch in range(_C_E // _LANES):
                    sl = pl.ds(ch * _LANES, _LANES)
                    acc = rows_v[r0, sl]
                    for j in range(1, _K):
                        acc = jnp.maximum(acc, rows_v[r0 + j, sl])
                    o_vmem[q, sl] = jnp.maximum(acc + base_vmem[q, sl], 0.0)

        pltpu.emit_pipeline(
            body,
            grid=(nrows // _W_PAIRS,),
            in_specs=[
                pl.BlockSpec((1, _W_IDX), lambda i: (0, i)),
                pl.BlockSpec((_W_PAIRS, _C_E), lambda i: (i, 0)),
            ],
            out_specs=[pl.BlockSpec((_W_PAIRS, _C_E), lambda i: (i, 0))],
            core_axis_name=("core", "subcore"),
            dimension_semantics=(pltpu.PARALLEL,),
        )(gidx_hbm, base_hbm, out_hbm)

    return k(mm_rows, gidx_flat, base_rows)


def kernel(inputs, w0, b0, w1, b1, w2, b2):
    xT = jnp.transpose(inputs, (0, 2, 1))
    w0T = jnp.transpose(w0)
    w1T = jnp.transpose(w1)
    w2T = jnp.transpose(w2)
    b0c = b0.reshape(_C_MID, 1)
    b0r = b0.reshape(1, _C_MID)
    bb = (b1 + b2).reshape(1, _C_E)
    halves = []
    nh = _N // 2
    for b in range(_B):
        mm_b, base_b, gidx_b = _tc_stage(
            inputs[b], xT[b], w0, w0T, w1T, w2T, b0c, b0r, bb)
        for h in range(2):
            halves.append(_sc_stage(
                mm_b,
                gidx_b[h * nh:(h + 1) * nh].reshape(1, nh * _K),
                base_b[h * nh:(h + 1) * nh]))
    y_rows = jnp.concatenate(halves).reshape(_B, _N, _C_E)
    y = jnp.transpose(y_rows, (0, 2, 1))
    return y.reshape(_B, _C_MID, 2 * _N)


# speedup vs baseline: 1.0732x; 1.0627x over previous
import functools

import jax
import jax.numpy as jnp
from jax import lax
from jax.experimental import pallas as pl
from jax.experimental.pallas import tpu as pltpu
from jax.experimental.pallas import tpu_sc as plsc

_B, _C_IN, _N = 4, 128, 2048
_C_MID = 128
_C_E = 256
_K = 16
_Q = 512
_R = _B * _N
_LANES = 16
_W_PAIRS = 8
_W_IDX = _W_PAIRS * _K

_NEG = -3.0e38


def _tc_body(xT_ref, x_ref, w0_ref, w0T_ref, w1T_ref, w2T_ref,
             b0c_ref, b0r_ref, bb_ref, mm_ref, base_ref, gidx_ref):
    x = x_ref[...]
    net = jnp.maximum(
        jnp.dot(w0_ref[...], x, preferred_element_type=jnp.float32)
        + b0c_ref[...], 0.0)
    netT = jnp.maximum(
        jnp.dot(xT_ref[...], w0T_ref[...], preferred_element_type=jnp.float32)
        + b0r_ref[...], 0.0)
    sq = jnp.sum(net * net, axis=0, keepdims=True)
    dots = jnp.dot(netT, net, preferred_element_type=jnp.float32)
    q0 = pl.program_id(0) * _Q
    qiota = lax.broadcasted_iota(jnp.int32, (_Q, _N), 0) + q0
    iota = lax.broadcasted_iota(jnp.int32, (_Q, _N), 1)
    s = jnp.where(iota == qiota, _NEG, 2.0 * dots - sq)
    mm = jnp.dot(netT, w1T_ref[...], preferred_element_type=jnp.float32)
    base = (jnp.dot(netT, w2T_ref[...], preferred_element_type=jnp.float32)
            + bb_ref[...] - mm)
    mm_ref[...] = mm
    base_ref[...] = base
    cols = []
    for t in range(_K):
        am = jnp.argmax(s, axis=1).astype(jnp.int32)[:, None]
        cols.append(am)
        if t < _K - 1:
            s = jnp.where(iota == am, _NEG, s)
    gidx_ref[...] = jnp.concatenate(cols, axis=1)


def _tc_stage(x_b, xT_b, w0, w0T, w1T, w2T, b0c, b0r, bb):
    return pl.pallas_call(
        _tc_body,
        grid=(_N // _Q,),
        in_specs=[
            pl.BlockSpec((_Q, _C_IN), lambda q: (q, 0)),
            pl.BlockSpec((_C_IN, _N), lambda q: (0, 0)),
            pl.BlockSpec((_C_MID, _C_IN), lambda q: (0, 0)),
            pl.BlockSpec((_C_IN, _C_MID), lambda q: (0, 0)),
            pl.BlockSpec((_C_MID, _C_E), lambda q: (0, 0)),
            pl.BlockSpec((_C_MID, _C_E), lambda q: (0, 0)),
            pl.BlockSpec((_C_MID, 1), lambda q: (0, 0)),
            pl.BlockSpec((1, _C_MID), lambda q: (0, 0)),
            pl.BlockSpec((1, _C_E), lambda q: (0, 0)),
        ],
        out_specs=[
            pl.BlockSpec((_Q, _C_E), lambda q: (q, 0)),
            pl.BlockSpec((_Q, _C_E), lambda q: (q, 0)),
            pl.BlockSpec((_Q, _K), lambda q: (q, 0)),
        ],
        out_shape=[
            jax.ShapeDtypeStruct((_N, _C_E), jnp.float32),
            jax.ShapeDtypeStruct((_N, _C_E), jnp.float32),
            jax.ShapeDtypeStruct((_N, _K), jnp.int32),
        ],
    )(xT_b, x_b, w0, w0T, w1T, w2T, b0c, b0r, bb)


def _sc_stage(mm_rows, gidx_flat, base_rows):
    mesh = plsc.VectorSubcoreMesh(core_axis_name="core",
                                  subcore_axis_name="subcore")

    nrows = base_rows.shape[0]

    @functools.partial(
        pl.kernel,
        out_type=jax.ShapeDtypeStruct((nrows, _C_E), jnp.float32),
        mesh=mesh,
        scratch_types=[pltpu.VMEM((_W_IDX, _C_E), jnp.float32)],
    )
    def k(mm_hbm, gidx_hbm, base_hbm, out_hbm, rows_v):
        def body(i_vmem, base_vmem, o_vmem):
            pltpu.sync_copy(mm_hbm.at[i_vmem.at[0]], rows_v)

            @pl.loop(0, _W_PAIRS)
            def _(q):
                r0 = q * _K
                for ch in range(_C_E // _LANES):
                    sl = pl.ds(ch * _LANES, _LANES)
                    acc = rows_v[r0, sl]
                    for j in range(1, _K):
                        acc = jnp.maximum(acc, rows_v[r0 + j, sl])
                    o_vmem[q, sl] = jnp.maximum(acc + base_vmem[q, sl], 0.0)

        pltpu.emit_pipeline(
            body,
            grid=(nrows // _W_PAIRS,),
            in_specs=[
                pl.BlockSpec((1, _W_IDX), lambda i: (0, i)),
                pl.BlockSpec((_W_PAIRS, _C_E), lambda i: (i, 0)),
            ],
            out_specs=[pl.BlockSpec((_W_PAIRS, _C_E), lambda i: (i, 0))],
            core_axis_name=("core", "subcore"),
            dimension_semantics=(pltpu.PARALLEL,),
        )(gidx_hbm, base_hbm, out_hbm)

    return k(mm_rows, gidx_flat, base_rows)


def kernel(inputs, w0, b0, w1, b1, w2, b2):
    xT = jnp.transpose(inputs, (0, 2, 1))
    w0T = jnp.transpose(w0)
    w1T = jnp.transpose(w1)
    w2T = jnp.transpose(w2)
    b0c = b0.reshape(_C_MID, 1)
    b0r = b0.reshape(1, _C_MID)
    bb = (b1 + b2).reshape(1, _C_E)
    ys = []
    for b in range(_B):
        mm_b, base_b, gidx_b = _tc_stage(
            inputs[b], xT[b], w0, w0T, w1T, w2T, b0c, b0r, bb)
        ys.append(_sc_stage(mm_b, gidx_b.reshape(1, _N * _K), base_b))
    y_rows = jnp.stack(ys)
    y = jnp.transpose(y_rows, (0, 2, 1))
    return y.reshape(_B, _C_MID, 2 * _N)
